# baseline (speedup 1.0000x reference)
import jax
import jax.numpy as jnp
from jax import lax
from jax.experimental import pallas as pl
from jax.experimental.pallas import tpu as pltpu

_ROWS = 4096
_COLS = 4096
_BLOCK = 256
_STEPS = 4
_QUARTER = _STEPS * _BLOCK


def _partials(x_ref, row_off):
    m = (x_ref[...] != 0.0).astype(jnp.float32)
    sel = lax.broadcasted_iota(jnp.int32, (2, _BLOCK), 0) == 0
    lane = lax.broadcasted_iota(jnp.int32, (2, _BLOCK), 1).astype(jnp.float32)
    w = jnp.where(sel, lane, 1.0)
    r = lax.dot_general(w, m, (((1,), (0,)), ((), ())),
                        preferred_element_type=jnp.float32)
    ri = r.astype(jnp.int32)
    col_ids = lax.broadcasted_iota(jnp.int32, (1, _COLS), 1)
    nnz = jnp.sum(ri[1:2])
    p_row = jnp.sum(ri[0:1]) + row_off * nnz
    p_col = jnp.sum(ri[1:2] * col_ids)
    return p_row, p_col


def _body(a_ref, b_ref, c_ref, d_ref, out_ref):
    i = pl.program_id(0)
    prs, pcs = 0, 0
    for k, ref in enumerate((a_ref, b_ref, c_ref, d_ref)):
        pr, pc = _partials(ref, k * _QUARTER + i * _BLOCK)
        prs += pr
        pcs += pc

    @pl.when(i == 0)
    def _init():
        out_ref[0] = 0
        out_ref[1] = 0

    out_ref[0] += prs
    out_ref[1] += pcs


def kernel(inputs):
    return pl.pallas_call(
        _body,
        grid=(_STEPS,),
        in_specs=[
            pl.BlockSpec((_BLOCK, _COLS), lambda i: (i, 0)),
            pl.BlockSpec((_BLOCK, _COLS), lambda i: (i + _STEPS, 0)),
            pl.BlockSpec((_BLOCK, _COLS), lambda i: (i + 2 * _STEPS, 0)),
            pl.BlockSpec((_BLOCK, _COLS), lambda i: (i + 3 * _STEPS, 0)),
        ],
        out_specs=pl.BlockSpec(memory_space=pltpu.SMEM),
        out_shape=jax.ShapeDtypeStruct((2,), jnp.int32),
    )(inputs, inputs, inputs, inputs)


# speedup vs baseline: 1.0194x; 1.0194x over previous
import jax
import jax.numpy as jnp
from jax import lax
from jax.experimental import pallas as pl
from jax.experimental.pallas import tpu as pltpu

_ROWS = 4096
_COLS = 4096
_BLOCK = 128
_STEPS = 8
_QUARTER = _STEPS * _BLOCK


def _partials(x_ref, row_off):
    m = (x_ref[...] != 0.0).astype(jnp.float32)
    sel = lax.broadcasted_iota(jnp.int32, (2, _BLOCK), 0) == 0
    lane = lax.broadcasted_iota(jnp.int32, (2, _BLOCK), 1).astype(jnp.float32)
    w = jnp.where(sel, lane, 1.0)
    r = lax.dot_general(w, m, (((1,), (0,)), ((), ())),
                        preferred_element_type=jnp.float32)
    ri = r.astype(jnp.int32)
    col_ids = lax.broadcasted_iota(jnp.int32, (1, _COLS), 1)
    nnz = jnp.sum(ri[1:2])
    p_row = jnp.sum(ri[0:1]) + row_off * nnz
    p_col = jnp.sum(ri[1:2] * col_ids)
    return p_row, p_col


def _body(a_ref, b_ref, c_ref, d_ref, out_ref):
    i = pl.program_id(0)
    prs, pcs = 0, 0
    for k, ref in enumerate((a_ref, b_ref, c_ref, d_ref)):
        pr, pc = _partials(ref, k * _QUARTER + i * _BLOCK)
        prs += pr
        pcs += pc

    @pl.when(i == 0)
    def _init():
        out_ref[0] = 0
        out_ref[1] = 0

    out_ref[0] += prs
    out_ref[1] += pcs


def kernel(inputs):
    return pl.pallas_call(
        _body,
        grid=(_STEPS,),
        in_specs=[
            pl.BlockSpec((_BLOCK, _COLS), lambda i: (i, 0)),
            pl.BlockSpec((_BLOCK, _COLS), lambda i: (i + _STEPS, 0)),
            pl.BlockSpec((_BLOCK, _COLS), lambda i: (i + 2 * _STEPS, 0)),
            pl.BlockSpec((_BLOCK, _COLS), lambda i: (i + 3 * _STEPS, 0)),
        ],
        out_specs=pl.BlockSpec(memory_space=pltpu.SMEM),
        out_shape=jax.ShapeDtypeStruct((2,), jnp.int32),
    )(inputs, inputs, inputs, inputs)


# speedup vs baseline: 1.0343x; 1.0146x over previous
import jax
import jax.numpy as jnp
from jax import lax
from jax.experimental import pallas as pl
from jax.experimental.pallas import tpu as pltpu

_ROWS = 4096
_COLS = 4096
_BLOCK = 256
_STEPS = 4
_QUARTER = _STEPS * _BLOCK


def _partials(x_ref, row_off):
    m = (x_ref[...] != 0.0).astype(jnp.float32)
    sel = lax.broadcasted_iota(jnp.int32, (2, _BLOCK), 0) == 0
    lane = lax.broadcasted_iota(jnp.int32, (2, _BLOCK), 1).astype(jnp.float32)
    w = jnp.where(sel, lane, 1.0)
    r = lax.dot_general(w, m, (((1,), (0,)), ((), ())),
                        preferred_element_type=jnp.float32)
    ri = r.astype(jnp.int32)
    col_ids = lax.broadcasted_iota(jnp.int32, (1, _COLS), 1)
    nnz = jnp.sum(ri[1:2])
    p_row = jnp.sum(ri[0:1]) + row_off * nnz
    p_col = jnp.sum(ri[1:2] * col_ids)
    return p_row, p_col


def _body(a_ref, b_ref, c_ref, d_ref, out_ref):
    i = pl.program_id(0)
    prs, pcs = 0, 0
    for k, ref in enumerate((a_ref, b_ref, c_ref, d_ref)):
        pr, pc = _partials(ref, k * _QUARTER + i * _BLOCK)
        prs += pr
        pcs += pc

    @pl.when(i == 0)
    def _init():
        out_ref[0] = 0
        out_ref[1] = 0

    out_ref[0] += prs
    out_ref[1] += pcs


def kernel(inputs):
    return pl.pallas_call(
        _body,
        grid=(_STEPS,),
        in_specs=[
            pl.BlockSpec((_BLOCK, _COLS), lambda i: (i, 0)),
            pl.BlockSpec((_BLOCK, _COLS), lambda i: (i + _STEPS, 0)),
            pl.BlockSpec((_BLOCK, _COLS), lambda i: (i + 2 * _STEPS, 0)),
            pl.BlockSpec((_BLOCK, _COLS), lambda i: (i + 3 * _STEPS, 0)),
        ],
        out_specs=pl.BlockSpec(memory_space=pltpu.SMEM),
        out_shape=jax.ShapeDtypeStruct((2,), jnp.int32),
    )(inputs, inputs, inputs, inputs)
